# baseline (speedup 1.0000x reference)
import functools

import jax
import jax.numpy as jnp
from jax.experimental import pallas as pl


def _proj_body(x_ref, w_ref, a1_ref, a2_ref, h_ref, s1_ref, s2_ref):
    x = x_ref[...]
    heads = w_ref.shape[0]
    s1s, s2s = [], []
    for hd in range(heads):
        h = jax.lax.dot_general(x, w_ref[hd], (((1,), (1,)), ((), ())),
                                preferred_element_type=jnp.float32)
        h_ref[hd] = h
        s1s.append(h @ a1_ref[hd])
        s2s.append(h @ a2_ref[hd])
    s1_ref[...] = jnp.stack(s1s, axis=1)
    s2_ref[...] = jnp.stack(s2s, axis=1)


@functools.partial(jax.jit, static_argnames=("blk",))
def _project(x, W, a, blk=400):
    n, in_f = x.shape
    heads, out_f, _ = W.shape
    a1 = a[:, :out_f]
    a2 = a[:, out_f:]
    grid = (n // blk,)
    h, s1, s2 = pl.pallas_call(
        _proj_body,
        grid=grid,
        in_specs=[
            pl.BlockSpec((blk, in_f), lambda i: (i, 0)),
            pl.BlockSpec((heads, out_f, in_f), lambda i: (0, 0, 0)),
            pl.BlockSpec((heads, out_f), lambda i: (0, 0)),
            pl.BlockSpec((heads, out_f), lambda i: (0, 0)),
        ],
        out_specs=[
            pl.BlockSpec((heads, blk, out_f), lambda i: (0, i, 0)),
            pl.BlockSpec((blk, heads), lambda i: (i, 0)),
            pl.BlockSpec((blk, heads), lambda i: (i, 0)),
        ],
        out_shape=[
            jax.ShapeDtypeStruct((heads, n, out_f), jnp.float32),
            jax.ShapeDtypeStruct((n, heads), jnp.float32),
            jax.ShapeDtypeStruct((n, heads), jnp.float32),
        ],
    )(x, W, a1, a2)
    return h, s1, s2


def kernel(x, edge_index, edge_weights, W, a):
    n = x.shape[0]
    heads = W.shape[0]
    h, s1, s2 = _project(x, W, a)
    src = edge_index[0]
    dst = edge_index[1]
    outs = []
    for hd in range(heads):
        e = s1[src, hd] + s2[dst, hd]
        e = jax.nn.leaky_relu(e, negative_slope=0.2)
        e = e * edge_weights
        m = jax.ops.segment_max(e, dst, num_segments=n)
        ex = jnp.exp(e - m[dst])
        den = jax.ops.segment_sum(ex, dst, num_segments=n)
        att = ex / (den[dst] + 1e-16)
        outs.append(jax.ops.segment_sum(h[hd][src] * att[:, None], dst,
                                        num_segments=n))
    return jnp.concatenate(outs, axis=-1)


# speedup vs baseline: 3.9281x; 3.9281x over previous
import functools

import jax
import jax.numpy as jnp
from jax import lax
from jax.experimental import pallas as pl
from jax.experimental.pallas import tpu as pltpu
from jax.experimental.pallas import tpu_sc as plsc

N = 10000
E = 160000
F = 256
H = 4
NW = 32
D = 320
NPAD = NW * D
DPAD = D
DBUF = DPAD + 16
CBUF = 2064
C = 2000
CPAD = 2048
G = 32
NEG = -2147483648


def _proj_body(x_ref, w_ref, a1_ref, a2_ref, h_ref, s1_ref, s2_ref):
    x = x_ref[...]
    heads = w_ref.shape[0]
    s1s, s2s = [], []
    for hd in range(heads):
        h = lax.dot_general(x, w_ref[hd], (((1,), (1,)), ((), ())),
                            preferred_element_type=jnp.float32)
        h_ref[hd] = h
        s1s.append(h @ a1_ref[hd])
        s2s.append(h @ a2_ref[hd])
    s1_ref[...] = jnp.stack(s1s, axis=1)
    s2_ref[...] = jnp.stack(s2s, axis=1)


@functools.partial(jax.jit, static_argnames=("blk",))
def _project(x, W, a, blk=400):
    n, in_f = x.shape
    heads, out_f, _ = W.shape
    a1 = a[:, :out_f]
    a2 = a[:, out_f:]
    grid = (n // blk,)
    h, s1, s2 = pl.pallas_call(
        _proj_body,
        grid=grid,
        in_specs=[
            pl.BlockSpec((blk, in_f), lambda i: (i, 0)),
            pl.BlockSpec((heads, out_f, in_f), lambda i: (0, 0, 0)),
            pl.BlockSpec((heads, out_f), lambda i: (0, 0)),
            pl.BlockSpec((heads, out_f), lambda i: (0, 0)),
        ],
        out_specs=[
            pl.BlockSpec((heads, blk, out_f), lambda i: (0, i, 0)),
            pl.BlockSpec((blk, heads), lambda i: (i, 0)),
            pl.BlockSpec((blk, heads), lambda i: (i, 0)),
        ],
        out_shape=[
            jax.ShapeDtypeStruct((heads, n, out_f), jnp.float32),
            jax.ShapeDtypeStruct((n, heads), jnp.float32),
            jax.ShapeDtypeStruct((n, heads), jnp.float32),
        ],
    )(x, W, a1, a2)
    return h, s1, s2


def _sortable(e):
    ib = plsc.bitcast(e, jnp.int32)
    return jnp.where(ib < 0, ib ^ 0x7FFFFFFF, ib)


def _unsortable(k):
    return plsc.bitcast(jnp.where(k < 0, k ^ 0x7FFFFFFF, k), jnp.float32)


def _edge_scores(s1_v, s2_v, src_c, dst_c, ew_c, g, base):
    sv = src_c[pl.ds(g * 16, 16)]
    dv = dst_c[pl.ds(g * 16, 16)]
    wv = ew_c[pl.ds(g * 16, 16)]
    e = plsc.load_gather(s1_v, [sv]) + plsc.load_gather(s2_v, [dv])
    e = jnp.where(e >= 0, e, 0.2 * e) * wv
    dl = dv - base
    own = (dl >= 0) & (dl < D)
    dlc = jnp.clip(dl, 0, DPAD - 1)
    return sv, e, dlc, own


def _count(own):
    return jnp.max(plsc.all_reduce_population_count(own))


def _sc_body(src_hbm, dst_hbm, ew_hbm, s1_hbm, s2_hbm, h_hbm, out_hbm,
             s1_v, s2_v, acc, mkey_v, mf_v, den_v, invd_v,
             src_c, dst_c, ew_c, dl_cc, key_cc, att_cc, src_cc,
             rowbuf, sem):
    wid = lax.axis_index("s") * 2 + lax.axis_index("c")
    base = wid * D
    lane0 = lax.iota(jnp.int32, 16) == 0

    zero16 = jnp.zeros((16,), jnp.float32)
    def zi(i, _):
        src_cc[pl.ds(i * 16, 16)] = jnp.zeros((16,), jnp.int32)
        return 0
    lax.fori_loop(0, CBUF // 16, zi, 0)

    nchunks = E // C
    ngroups = C // 16

    for hd in range(H):
        pltpu.sync_copy(s1_hbm.at[hd], s1_v)
        pltpu.sync_copy(s2_hbm.at[hd], s2_v)

        def zseg(i, _):
            mkey_v[pl.ds(i * 16, 16)] = jnp.full((16,), NEG, jnp.int32)
            den_v[pl.ds(i * 16, 16)] = zero16
            return 0
        lax.fori_loop(0, DBUF // 16, zseg, 0)

        def zacc(i, _):
            for j in range(F // 16):
                acc[i, pl.ds(j * 16, 16)] = zero16
            return 0
        lax.fori_loop(0, D, zacc, 0)

        def p1_chunk(c, _):
            pltpu.sync_copy(src_hbm.at[pl.ds(c * C, C)], src_c)
            pltpu.sync_copy(dst_hbm.at[pl.ds(c * C, C)], dst_c)
            pltpu.sync_copy(ew_hbm.at[pl.ds(c * C, C)], ew_c)

            def p1_g(g, cnt):
                _, e, dlc, own = _edge_scores(s1_v, s2_v, src_c, dst_c, ew_c,
                                              g, base)
                plsc.store_compressed(dl_cc.at[pl.ds(cnt, 16)], dlc, mask=own)
                plsc.store_compressed(key_cc.at[pl.ds(cnt, 16)],
                                      _sortable(e), mask=own)
                return cnt + _count(own)
            k = lax.fori_loop(0, ngroups, p1_g, 0)

            def p1_s(j16, _):
                dlv = dl_cc[pl.ds(j16 * 16, 16)]
                kv = key_cc[pl.ds(j16 * 16, 16)]
                for l in range(16):
                    @pl.when(j16 * 16 + l < k)
                    def _():
                        d = dlv[l]
                        seg = mkey_v[pl.ds(d, 16)]
                        mkey_v[pl.ds(d, 16)] = jnp.where(
                            lane0, jnp.maximum(seg, kv[l]), seg)
                return 0
            lax.fori_loop(0, (k + 15) // 16, p1_s, 0)
            return 0
        lax.fori_loop(0, nchunks, p1_chunk, 0)

        def dec(i, _):
            mf_v[pl.ds(i * 16, 16)] = _unsortable(mkey_v[pl.ds(i * 16, 16)])
            return 0
        lax.fori_loop(0, DBUF // 16, dec, 0)

        def p2_chunk(c, _):
            pltpu.sync_copy(src_hbm.at[pl.ds(c * C, C)], src_c)
            pltpu.sync_copy(dst_hbm.at[pl.ds(c * C, C)], dst_c)
            pltpu.sync_copy(ew_hbm.at[pl.ds(c * C, C)], ew_c)

            def p2_g(g, cnt):
                _, e, dlc, own = _edge_scores(s1_v, s2_v, src_c, dst_c, ew_c,
                                              g, base)
                ex = jnp.exp(e - plsc.load_gather(mf_v, [dlc]))
                plsc.store_compressed(dl_cc.at[pl.ds(cnt, 16)], dlc, mask=own)
                plsc.store_compressed(att_cc.at[pl.ds(cnt, 16)], ex, mask=own)
                return cnt + _count(own)
            k = lax.fori_loop(0, ngroups, p2_g, 0)

            def p2_s(j16, _):
                dlv = dl_cc[pl.ds(j16 * 16, 16)]
                ev = att_cc[pl.ds(j16 * 16, 16)]
                for l in range(16):
                    @pl.when(j16 * 16 + l < k)
                    def _():
                        d = dlv[l]
                        seg = den_v[pl.ds(d, 16)]
                        den_v[pl.ds(d, 16)] = jnp.where(
                            lane0, seg + ev[l], seg)
                return 0
            lax.fori_loop(0, (k + 15) // 16, p2_s, 0)
            return 0
        lax.fori_loop(0, nchunks, p2_chunk, 0)

        def inv(i, _):
            invd_v[pl.ds(i * 16, 16)] = 1.0 / (den_v[pl.ds(i * 16, 16)]
                                               + 1e-16)
            return 0
        lax.fori_loop(0, DBUF // 16, inv, 0)

        def p3_chunk(c, _):
            pltpu.sync_copy(src_hbm.at[pl.ds(c * C, C)], src_c)
            pltpu.sync_copy(dst_hbm.at[pl.ds(c * C, C)], dst_c)
            pltpu.sync_copy(ew_hbm.at[pl.ds(c * C, C)], ew_c)

            def p3_g(g, cnt):
                sv, e, dlc, own = _edge_scores(s1_v, s2_v, src_c, dst_c, ew_c,
                                               g, base)
                ex = jnp.exp(e - plsc.load_gather(mf_v, [dlc]))
                att = ex * plsc.load_gather(invd_v, [dlc])
                plsc.store_compressed(dl_cc.at[pl.ds(cnt, 16)], dlc, mask=own)
                plsc.store_compressed(att_cc.at[pl.ds(cnt, 16)], att, mask=own)
                plsc.store_compressed(src_cc.at[pl.ds(cnt, 16)], sv, mask=own)
                return cnt + _count(own)
            k = lax.fori_loop(0, ngroups, p3_g, 0)

            def p3_grp(gi, _):
                off = gi * G
                pltpu.async_copy(h_hbm.at[hd].at[src_cc.at[pl.ds(off, G)]],
                                 rowbuf, sem).wait()

                def p3_r(r, _):
                    a = att_cc[pl.ds(off + r, 16)][0]
                    d = dl_cc[pl.ds(off + r, 16)][0]
                    for j in range(F // 16):
                        v = rowbuf[r, pl.ds(j * 16, 16)] * a
                        plsc.addupdate(acc.at[d, pl.ds(j * 16, 16)], v)
                    return 0
                lax.fori_loop(0, jnp.minimum(G, k - off), p3_r, 0)
                return 0
            lax.fori_loop(0, (k + G - 1) // G, p3_grp, 0)
            return 0
        lax.fori_loop(0, nchunks, p3_chunk, 0)

        pltpu.sync_copy(acc, out_hbm.at[hd].at[pl.ds(base, D)])


@jax.jit
def _edge_phase(src, dst, ew, s1, s2, h):
    mesh = plsc.VectorSubcoreMesh(core_axis_name="c", subcore_axis_name="s")
    f = functools.partial(
        pl.kernel,
        out_type=jax.ShapeDtypeStruct((H, NPAD, F), jnp.float32),
        scratch_types=[
            pltpu.VMEM((N,), jnp.float32),
            pltpu.VMEM((N,), jnp.float32),
            pltpu.VMEM((D, F), jnp.float32),
            pltpu.VMEM((DBUF,), jnp.int32),
            pltpu.VMEM((DBUF,), jnp.float32),
            pltpu.VMEM((DBUF,), jnp.float32),
            pltpu.VMEM((DBUF,), jnp.float32),
            pltpu.VMEM((C,), jnp.int32),
            pltpu.VMEM((C,), jnp.int32),
            pltpu.VMEM((C,), jnp.float32),
            pltpu.VMEM((CBUF,), jnp.int32),
            pltpu.VMEM((CBUF,), jnp.int32),
            pltpu.VMEM((CBUF,), jnp.float32),
            pltpu.VMEM((CBUF,), jnp.int32),
            pltpu.VMEM((G, F), jnp.float32),
            pltpu.SemaphoreType.DMA,
        ],
        compiler_params=pltpu.CompilerParams(needs_layout_passes=False),
        mesh=mesh,
    )(_sc_body)
    return f(src, dst, ew, s1, s2, h)


def kernel(x, edge_index, edge_weights, W, a):
    h, s1, s2 = _project(x, W, a)
    out = _edge_phase(edge_index[0], edge_index[1], edge_weights,
                      s1.T.copy(), s2.T.copy(), h)
    return out[:, :N].transpose(1, 0, 2).reshape(N, H * F)


# speedup vs baseline: 5.8270x; 1.4834x over previous
import functools

import jax
import jax.numpy as jnp
from jax import lax
from jax.experimental import pallas as pl
from jax.experimental.pallas import tpu as pltpu
from jax.experimental.pallas import tpu_sc as plsc

N = 10000
E = 160000
F = 256
H = 4
NW = 32
D = 320
NPAD = NW * D
DBUF = D + 16
C = 1600
CBUF = C + 16
G = 16
NEG = -2147483648


def _proj_body(x_ref, w_ref, a1_ref, a2_ref, h_ref, s1_ref, s2_ref):
    x = x_ref[...]
    heads = w_ref.shape[0]
    s1s, s2s = [], []
    for hd in range(heads):
        h = lax.dot_general(x, w_ref[hd], (((1,), (1,)), ((), ())),
                            preferred_element_type=jnp.float32)
        h_ref[hd] = h
        s1s.append(h @ a1_ref[hd])
        s2s.append(h @ a2_ref[hd])
    s1_ref[...] = jnp.stack(s1s, axis=1)
    s2_ref[...] = jnp.stack(s2s, axis=1)


@functools.partial(jax.jit, static_argnames=("blk",))
def _project(x, W, a, blk=400):
    n, in_f = x.shape
    heads, out_f, _ = W.shape
    a1 = a[:, :out_f]
    a2 = a[:, out_f:]
    grid = (n // blk,)
    h, s1, s2 = pl.pallas_call(
        _proj_body,
        grid=grid,
        in_specs=[
            pl.BlockSpec((blk, in_f), lambda i: (i, 0)),
            pl.BlockSpec((heads, out_f, in_f), lambda i: (0, 0, 0)),
            pl.BlockSpec((heads, out_f), lambda i: (0, 0)),
            pl.BlockSpec((heads, out_f), lambda i: (0, 0)),
        ],
        out_specs=[
            pl.BlockSpec((heads, blk, out_f), lambda i: (0, i, 0)),
            pl.BlockSpec((blk, heads), lambda i: (i, 0)),
            pl.BlockSpec((blk, heads), lambda i: (i, 0)),
        ],
        out_shape=[
            jax.ShapeDtypeStruct((heads, n, out_f), jnp.float32),
            jax.ShapeDtypeStruct((n, heads), jnp.float32),
            jax.ShapeDtypeStruct((n, heads), jnp.float32),
        ],
    )(x, W, a1, a2)
    return h, s1, s2


def _sortable(e):
    ib = plsc.bitcast(e, jnp.int32)
    return jnp.where(ib < 0, ib ^ 0x7FFFFFFF, ib)


def _unsortable(k):
    return plsc.bitcast(jnp.where(k < 0, k ^ 0x7FFFFFFF, k), jnp.float32)


def _sc_body(src_hbm, dst_hbm, ew_hbm, s1_hbm, s2_hbm, h_hbm, out_hbm,
             s1_v, s2_v, acc, mkey_v, mf_v, den_v, invd_v,
             src_c, dst_c, ew_c, dl_cc, key_cc, att_cc, src_cc,
             rowbuf, csem, gsem):
    wid = lax.axis_index("s") * 2 + lax.axis_index("c")
    base = wid * D
    lane0 = lax.iota(jnp.int32, 16) == 0
    zero16 = jnp.zeros((16,), jnp.float32)

    nchunks = E // C
    ngroups = C // 16

    def issue_chunk(c, slot):
        off = slot * C
        pltpu.async_copy(src_hbm.at[pl.ds(c * C, C)],
                         src_c.at[pl.ds(off, C)], csem)
        pltpu.async_copy(dst_hbm.at[pl.ds(c * C, C)],
                         dst_c.at[pl.ds(off, C)], csem)
        pltpu.async_copy(ew_hbm.at[pl.ds(c * C, C)],
                         ew_c.at[pl.ds(off, C)], csem)

    def wait_chunk(slot):
        off = slot * C
        pltpu.make_async_copy(src_hbm.at[pl.ds(0, C)],
                              src_c.at[pl.ds(off, C)], csem).wait()
        pltpu.make_async_copy(dst_hbm.at[pl.ds(0, C)],
                              dst_c.at[pl.ds(off, C)], csem).wait()
        pltpu.make_async_copy(ew_hbm.at[pl.ds(0, C)],
                              ew_c.at[pl.ds(off, C)], csem).wait()

    def edge_scores(g, slot):
        off = slot * C + g * 16
        sv = src_c[pl.ds(off, 16)]
        dv = dst_c[pl.ds(off, 16)]
        wv = ew_c[pl.ds(off, 16)]
        e = plsc.load_gather(s1_v, [sv]) + plsc.load_gather(s2_v, [dv])
        e = jnp.where(e >= 0, e, 0.2 * e) * wv
        dl = dv - base
        own = (dl >= 0) & (dl < D)
        dlc = jnp.clip(dl, 0, D - 1)
        return sv, e, dlc, own

    def count(own):
        return jnp.max(plsc.all_reduce_population_count(own))

    def zi(i, _):
        src_cc[pl.ds(i * 16, 16)] = jnp.zeros((16,), jnp.int32)
        return 0
    lax.fori_loop(0, CBUF // 16, zi, 0)

    for hd in range(H):
        pltpu.sync_copy(s1_hbm.at[hd], s1_v)
        pltpu.sync_copy(s2_hbm.at[hd], s2_v)

        def zseg(i, _):
            mkey_v[pl.ds(i * 16, 16)] = jnp.full((16,), NEG, jnp.int32)
            den_v[pl.ds(i * 16, 16)] = zero16
            return 0
        lax.fori_loop(0, DBUF // 16, zseg, 0)

        def zacc(i, _):
            for j in range(F // 16):
                acc[i, pl.ds(j * 16, 16)] = zero16
            return 0
        lax.fori_loop(0, D, zacc, 0)

        issue_chunk(0, 0)

        def p1_chunk(c, _):
            slot = c & 1
            wait_chunk(slot)

            @pl.when(c + 1 < nchunks)
            def _():
                issue_chunk(c + 1, (c + 1) & 1)

            def p1_g(g, cnt):
                _, e, dlc, own = edge_scores(g, slot)
                plsc.store_compressed(dl_cc.at[pl.ds(cnt, 16)], dlc, mask=own)
                plsc.store_compressed(key_cc.at[pl.ds(cnt, 16)],
                                      _sortable(e), mask=own)
                return cnt + count(own)
            k = lax.fori_loop(0, ngroups, p1_g, 0)

            def p1_s(j16, _):
                dlv = dl_cc[pl.ds(j16 * 16, 16)]
                kv = key_cc[pl.ds(j16 * 16, 16)]
                for l in range(16):
                    @pl.when(j16 * 16 + l < k)
                    def _():
                        d = dlv[l]
                        seg = mkey_v[pl.ds(d, 16)]
                        mkey_v[pl.ds(d, 16)] = jnp.where(
                            lane0, jnp.maximum(seg, kv[l]), seg)
                return 0
            lax.fori_loop(0, (k + 15) // 16, p1_s, 0)
            return 0
        lax.fori_loop(0, nchunks, p1_chunk, 0)

        def dec(i, _):
            mf_v[pl.ds(i * 16, 16)] = _unsortable(mkey_v[pl.ds(i * 16, 16)])
            return 0
        lax.fori_loop(0, DBUF // 16, dec, 0)

        issue_chunk(0, 0)

        def p2_chunk(c, _):
            slot = c & 1
            wait_chunk(slot)

            @pl.when(c + 1 < nchunks)
            def _():
                issue_chunk(c + 1, (c + 1) & 1)

            def p2_g(g, cnt):
                _, e, dlc, own = edge_scores(g, slot)
                ex = jnp.exp(e - plsc.load_gather(mf_v, [dlc]))
                plsc.store_compressed(dl_cc.at[pl.ds(cnt, 16)], dlc, mask=own)
                plsc.store_compressed(att_cc.at[pl.ds(cnt, 16)], ex, mask=own)
                return cnt + count(own)
            k = lax.fori_loop(0, ngroups, p2_g, 0)

            def p2_s(j16, _):
                dlv = dl_cc[pl.ds(j16 * 16, 16)]
                ev = att_cc[pl.ds(j16 * 16, 16)]
                for l in range(16):
                    @pl.when(j16 * 16 + l < k)
                    def _():
                        d = dlv[l]
                        seg = den_v[pl.ds(d, 16)]
                        den_v[pl.ds(d, 16)] = jnp.where(
                            lane0, seg + ev[l], seg)
                return 0
            lax.fori_loop(0, (k + 15) // 16, p2_s, 0)
            return 0
        lax.fori_loop(0, nchunks, p2_chunk, 0)

        def inv(i, _):
            invd_v[pl.ds(i * 16, 16)] = 1.0 / (den_v[pl.ds(i * 16, 16)]
                                               + 1e-16)
            return 0
        lax.fori_loop(0, DBUF // 16, inv, 0)

        issue_chunk(0, 0)

        def p3_chunk(c, _):
            slot = c & 1
            wait_chunk(slot)

            @pl.when(c + 1 < nchunks)
            def _():
                issue_chunk(c + 1, (c + 1) & 1)

            def p3_g(g, cnt):
                sv, e, dlc, own = edge_scores(g, slot)
                ex = jnp.exp(e - plsc.load_gather(mf_v, [dlc]))
                att = ex * plsc.load_gather(invd_v, [dlc])
                plsc.store_compressed(dl_cc.at[pl.ds(cnt, 16)], dlc, mask=own)
                plsc.store_compressed(att_cc.at[pl.ds(cnt, 16)], att,
                                      mask=own)
                plsc.store_compressed(src_cc.at[pl.ds(cnt, 16)], sv, mask=own)
                return cnt + count(own)
            k = lax.fori_loop(0, ngroups, p3_g, 0)
            ng = (k + G - 1) // G

            def issue_rows(gi):
                pltpu.async_copy(
                    h_hbm.at[hd].at[src_cc.at[pl.ds(gi * G, G)]],
                    rowbuf.at[pl.ds((gi & 1) * G, G)], gsem)

            @pl.when(ng > 0)
            def _():
                issue_rows(0)

            def p3_grp(gi, _):
                gslot = gi & 1
                pltpu.make_async_copy(
                    h_hbm.at[hd].at[src_cc.at[pl.ds(0, G)]],
                    rowbuf.at[pl.ds(gslot * G, G)], gsem).wait()

                @pl.when(gi + 1 < ng)
                def _():
                    issue_rows(gi + 1)

                def p3_r(r, _):
                    off = gi * G + r
                    a = att_cc[pl.ds(off, 16)][0]
                    d = dl_cc[pl.ds(off, 16)][0]
                    row = gslot * G + r
                    for j in range(F // 16):
                        v = rowbuf[row, pl.ds(j * 16, 16)] * a
                        plsc.addupdate(acc.at[d, pl.ds(j * 16, 16)], v)
                    return 0
                lax.fori_loop(0, jnp.minimum(G, k - gi * G), p3_r, 0)
                return 0
            lax.fori_loop(0, ng, p3_grp, 0)
            return 0
        lax.fori_loop(0, nchunks, p3_chunk, 0)

        pltpu.sync_copy(acc, out_hbm.at[hd].at[pl.ds(base, D)])


@jax.jit
def _edge_phase(src, dst, ew, s1, s2, h):
    mesh = plsc.VectorSubcoreMesh(core_axis_name="c", subcore_axis_name="s")
    f = functools.partial(
        pl.kernel,
        out_type=jax.ShapeDtypeStruct((H, NPAD, F), jnp.float32),
        scratch_types=[
            pltpu.VMEM((N,), jnp.float32),
            pltpu.VMEM((N,), jnp.float32),
            pltpu.VMEM((D, F), jnp.float32),
            pltpu.VMEM((DBUF,), jnp.int32),
            pltpu.VMEM((DBUF,), jnp.float32),
            pltpu.VMEM((DBUF,), jnp.float32),
            pltpu.VMEM((DBUF,), jnp.float32),
            pltpu.VMEM((2 * C,), jnp.int32),
            pltpu.VMEM((2 * C,), jnp.int32),
            pltpu.VMEM((2 * C,), jnp.float32),
            pltpu.VMEM((CBUF,), jnp.int32),
            pltpu.VMEM((CBUF,), jnp.int32),
            pltpu.VMEM((CBUF,), jnp.float32),
            pltpu.VMEM((CBUF,), jnp.int32),
            pltpu.VMEM((2 * G, F), jnp.float32),
            pltpu.SemaphoreType.DMA,
            pltpu.SemaphoreType.DMA,
        ],
        compiler_params=pltpu.CompilerParams(needs_layout_passes=False),
        mesh=mesh,
    )(_sc_body)
    return f(src, dst, ew, s1, s2, h)


def kernel(x, edge_index, edge_weights, W, a):
    h, s1, s2 = _project(x, W, a)
    out = _edge_phase(edge_index[0], edge_index[1], edge_weights,
                      s1.T.copy(), s2.T.copy(), h)
    return out[:, :N].transpose(1, 0, 2).reshape(N, H * F)


# speedup vs baseline: 7.2223x; 1.2394x over previous
import functools

import jax
import jax.numpy as jnp
from jax import lax
from jax.experimental import pallas as pl
from jax.experimental.pallas import tpu as pltpu
from jax.experimental.pallas import tpu_sc as plsc

N = 10000
E = 160000
F = 256
H = 4
NW = 32
D = 320
NPAD = NW * D
DBUF = D + 16
C = 1600
CBUF = C + 16
G = 16
NEG = -2147483648


def _proj_body(x_ref, w_ref, a1_ref, a2_ref, h_ref, s1_ref, s2_ref):
    x = x_ref[...]
    heads = w_ref.shape[0]
    s1s, s2s = [], []
    for hd in range(heads):
        h = lax.dot_general(x, w_ref[hd], (((1,), (1,)), ((), ())),
                            preferred_element_type=jnp.float32)
        h_ref[hd] = h
        s1s.append(h @ a1_ref[hd])
        s2s.append(h @ a2_ref[hd])
    s1_ref[...] = jnp.stack(s1s, axis=1)
    s2_ref[...] = jnp.stack(s2s, axis=1)


@functools.partial(jax.jit, static_argnames=("blk",))
def _project(x, W, a, blk=400):
    n, in_f = x.shape
    heads, out_f, _ = W.shape
    a1 = a[:, :out_f]
    a2 = a[:, out_f:]
    grid = (n // blk,)
    h, s1, s2 = pl.pallas_call(
        _proj_body,
        grid=grid,
        in_specs=[
            pl.BlockSpec((blk, in_f), lambda i: (i, 0)),
            pl.BlockSpec((heads, out_f, in_f), lambda i: (0, 0, 0)),
            pl.BlockSpec((heads, out_f), lambda i: (0, 0)),
            pl.BlockSpec((heads, out_f), lambda i: (0, 0)),
        ],
        out_specs=[
            pl.BlockSpec((heads, blk, out_f), lambda i: (0, i, 0)),
            pl.BlockSpec((blk, heads), lambda i: (i, 0)),
            pl.BlockSpec((blk, heads), lambda i: (i, 0)),
        ],
        out_shape=[
            jax.ShapeDtypeStruct((heads, n, out_f), jnp.float32),
            jax.ShapeDtypeStruct((n, heads), jnp.float32),
            jax.ShapeDtypeStruct((n, heads), jnp.float32),
        ],
    )(x, W, a1, a2)
    return h, s1, s2


def _sortable(e):
    ib = plsc.bitcast(e, jnp.int32)
    return jnp.where(ib < 0, ib ^ 0x7FFFFFFF, ib)


def _unsortable(k):
    return plsc.bitcast(jnp.where(k < 0, k ^ 0x7FFFFFFF, k), jnp.float32)


def _sc_body(src_hbm, dst_hbm, ew_hbm, s1_hbm, s2_hbm, h_hbm, out_hbm,
             s1_v, s2_v, acc, mkey_v, mf_v, den_v, invd_v,
             src_c, dst_c, ew_c, dl_cc, key_cc, att_cc, src_cc,
             rowbuf, csem, gsem):
    wid = lax.axis_index("s") * 2 + lax.axis_index("c")
    base = wid * D
    lane0 = lax.iota(jnp.int32, 16) == 0
    zero16 = jnp.zeros((16,), jnp.float32)

    nchunks = E // C
    ngroups = C // 16

    def issue_chunk(c, slot):
        off = slot * C
        pltpu.async_copy(src_hbm.at[pl.ds(c * C, C)],
                         src_c.at[pl.ds(off, C)], csem)
        pltpu.async_copy(dst_hbm.at[pl.ds(c * C, C)],
                         dst_c.at[pl.ds(off, C)], csem)
        pltpu.async_copy(ew_hbm.at[pl.ds(c * C, C)],
                         ew_c.at[pl.ds(off, C)], csem)

    def wait_chunk(slot):
        off = slot * C
        pltpu.make_async_copy(src_hbm.at[pl.ds(0, C)],
                              src_c.at[pl.ds(off, C)], csem).wait()
        pltpu.make_async_copy(dst_hbm.at[pl.ds(0, C)],
                              dst_c.at[pl.ds(off, C)], csem).wait()
        pltpu.make_async_copy(ew_hbm.at[pl.ds(0, C)],
                              ew_c.at[pl.ds(off, C)], csem).wait()

    def edge_scores(g, slot):
        off = slot * C + g * 16
        sv = src_c[pl.ds(off, 16)]
        dv = dst_c[pl.ds(off, 16)]
        wv = ew_c[pl.ds(off, 16)]
        e = plsc.load_gather(s1_v, [sv]) + plsc.load_gather(s2_v, [dv])
        e = jnp.where(e >= 0, e, 0.2 * e) * wv
        dl = dv - base
        own = (dl >= 0) & (dl < D)
        dlc = jnp.clip(dl, 0, D - 1)
        return sv, e, dlc, own

    def count(own):
        return jnp.max(plsc.all_reduce_population_count(own))

    def zi(i, _):
        src_cc[pl.ds(i * 16, 16)] = jnp.zeros((16,), jnp.int32)
        return 0
    lax.fori_loop(0, CBUF // 16, zi, 0)

    for hd in range(H):
        pltpu.sync_copy(s1_hbm.at[hd], s1_v)
        pltpu.sync_copy(s2_hbm.at[hd], s2_v)

        def zseg(i, _):
            mkey_v[pl.ds(i * 16, 16)] = jnp.full((16,), NEG, jnp.int32)
            den_v[pl.ds(i * 16, 16)] = zero16
            return 0
        lax.fori_loop(0, DBUF // 16, zseg, 0)

        def zacc(i, _):
            for j in range(F // 16):
                acc[i, pl.ds(j * 16, 16)] = zero16
            return 0
        lax.fori_loop(0, D, zacc, 0)

        issue_chunk(0, 0)

        def p1_chunk(c, _):
            slot = c & 1
            wait_chunk(slot)

            @pl.when(c + 1 < nchunks)
            def _():
                issue_chunk(c + 1, (c + 1) & 1)

            def p1_g(g, cnt):
                _, e, dlc, own = edge_scores(g, slot)
                plsc.store_compressed(dl_cc.at[pl.ds(cnt, 16)], dlc, mask=own)
                plsc.store_compressed(key_cc.at[pl.ds(cnt, 16)],
                                      _sortable(e), mask=own)
                return cnt + count(own)
            k = lax.fori_loop(0, ngroups, p1_g, 0)

            def p1_s(j16, _):
                dlv = dl_cc[pl.ds(j16 * 16, 16)]
                kv = key_cc[pl.ds(j16 * 16, 16)]
                for l in range(16):
                    @pl.when(j16 * 16 + l < k)
                    def _():
                        d = dlv[l]
                        seg = mkey_v[pl.ds(d, 16)]
                        mkey_v[pl.ds(d, 16)] = jnp.where(
                            lane0, jnp.maximum(seg, kv[l]), seg)
                return 0
            lax.fori_loop(0, (k + 15) // 16, p1_s, 0)
            return 0
        lax.fori_loop(0, nchunks, p1_chunk, 0)

        def dec(i, _):
            mf_v[pl.ds(i * 16, 16)] = _unsortable(mkey_v[pl.ds(i * 16, 16)])
            return 0
        lax.fori_loop(0, DBUF // 16, dec, 0)

        issue_chunk(0, 0)

        def p3_chunk(c, _):
            slot = c & 1
            wait_chunk(slot)

            @pl.when(c + 1 < nchunks)
            def _():
                issue_chunk(c + 1, (c + 1) & 1)

            def p3_g(g, cnt):
                sv, e, dlc, own = edge_scores(g, slot)
                ex = jnp.exp(e - plsc.load_gather(mf_v, [dlc]))
                plsc.store_compressed(dl_cc.at[pl.ds(cnt, 16)], dlc, mask=own)
                plsc.store_compressed(att_cc.at[pl.ds(cnt, 16)], ex, mask=own)
                plsc.store_compressed(src_cc.at[pl.ds(cnt, 16)], sv, mask=own)
                return cnt + count(own)
            k = lax.fori_loop(0, ngroups, p3_g, 0)

            def p3_den(j16, _):
                dlv = dl_cc[pl.ds(j16 * 16, 16)]
                ev = att_cc[pl.ds(j16 * 16, 16)]
                for l in range(16):
                    @pl.when(j16 * 16 + l < k)
                    def _():
                        d = dlv[l]
                        seg = den_v[pl.ds(d, 16)]
                        den_v[pl.ds(d, 16)] = jnp.where(
                            lane0, seg + ev[l], seg)
                return 0
            lax.fori_loop(0, (k + 15) // 16, p3_den, 0)
            ng = (k + G - 1) // G

            def issue_rows(gi):
                pltpu.async_copy(
                    h_hbm.at[hd].at[src_cc.at[pl.ds(gi * G, G)]],
                    rowbuf.at[pl.ds((gi & 1) * G, G)], gsem)

            @pl.when(ng > 0)
            def _():
                issue_rows(0)

            def p3_grp(gi, _):
                gslot = gi & 1
                pltpu.make_async_copy(
                    h_hbm.at[hd].at[src_cc.at[pl.ds(0, G)]],
                    rowbuf.at[pl.ds(gslot * G, G)], gsem).wait()

                @pl.when(gi + 1 < ng)
                def _():
                    issue_rows(gi + 1)

                def p3_r(r, _):
                    off = gi * G + r
                    a = att_cc[pl.ds(off, 16)][0]
                    d = dl_cc[pl.ds(off, 16)][0]
                    row = gslot * G + r
                    for j in range(F // 16):
                        v = rowbuf[row, pl.ds(j * 16, 16)] * a
                        plsc.addupdate(acc.at[d, pl.ds(j * 16, 16)], v)
                    return 0
                lax.fori_loop(0, jnp.minimum(G, k - gi * G), p3_r, 0)
                return 0
            lax.fori_loop(0, ng, p3_grp, 0)
            return 0
        lax.fori_loop(0, nchunks, p3_chunk, 0)

        def inv(i, _):
            invd_v[pl.ds(i * 16, 16)] = 1.0 / (den_v[pl.ds(i * 16, 16)]
                                               + 1e-16)
            return 0
        lax.fori_loop(0, DBUF // 16, inv, 0)

        def scale(i, _):
            s = invd_v[pl.ds(i, 16)][0]
            for j in range(F // 16):
                acc[i, pl.ds(j * 16, 16)] = acc[i, pl.ds(j * 16, 16)] * s
            return 0
        lax.fori_loop(0, D, scale, 0)

        pltpu.sync_copy(acc, out_hbm.at[hd].at[pl.ds(base, D)])


@jax.jit
def _edge_phase(src, dst, ew, s1, s2, h):
    mesh = plsc.VectorSubcoreMesh(core_axis_name="c", subcore_axis_name="s")
    f = functools.partial(
        pl.kernel,
        out_type=jax.ShapeDtypeStruct((H, NPAD, F), jnp.float32),
        scratch_types=[
            pltpu.VMEM((N,), jnp.float32),
            pltpu.VMEM((N,), jnp.float32),
            pltpu.VMEM((D, F), jnp.float32),
            pltpu.VMEM((DBUF,), jnp.int32),
            pltpu.VMEM((DBUF,), jnp.float32),
            pltpu.VMEM((DBUF,), jnp.float32),
            pltpu.VMEM((DBUF,), jnp.float32),
            pltpu.VMEM((2 * C,), jnp.int32),
            pltpu.VMEM((2 * C,), jnp.int32),
            pltpu.VMEM((2 * C,), jnp.float32),
            pltpu.VMEM((CBUF,), jnp.int32),
            pltpu.VMEM((CBUF,), jnp.int32),
            pltpu.VMEM((CBUF,), jnp.float32),
            pltpu.VMEM((CBUF,), jnp.int32),
            pltpu.VMEM((2 * G, F), jnp.float32),
            pltpu.SemaphoreType.DMA,
            pltpu.SemaphoreType.DMA,
        ],
        compiler_params=pltpu.CompilerParams(needs_layout_passes=False),
        mesh=mesh,
    )(_sc_body)
    return f(src, dst, ew, s1, s2, h)


def kernel(x, edge_index, edge_weights, W, a):
    h, s1, s2 = _project(x, W, a)
    out = _edge_phase(edge_index[0], edge_index[1], edge_weights,
                      s1.T.copy(), s2.T.copy(), h)
    return out[:, :N].transpose(1, 0, 2).reshape(N, H * F)
